# Initial kernel scaffold; baseline (speedup 1.0000x reference)
import jax
import jax.numpy as jnp
from jax.experimental import pallas as pl


def kernel(x, full_ei, edge_index_batch, edge_attr_batch, W_l1, b_l1, W_r1, W_l2, b_l2, W_r2, gamma, beta, W1, b1, W2, b2, W3, b3):
    raise NotImplementedError("write your pallas kernel here")


# speedup vs baseline: 3.3409x; 3.3409x over previous
import functools

import jax
import jax.numpy as jnp
from jax import lax
from jax.experimental import pallas as pl
from jax.experimental.pallas import tpu as pltpu
from jax.experimental.pallas import tpu_sc as plsc

N = 10000
E = 320000
D = 128
H = 96
EA = 16

NC = 2
NS = 16
NW = NC * NS
CH = 80
EPW = E // NW
N_PAD = ((N + 1 + NS * 8 - 1) // (NS * 8)) * (NS * 8)
RPT = N_PAD // NS


def _segsum(x, src, dst, d, with_count):
    mesh = plsc.VectorSubcoreMesh(core_axis_name="c", subcore_axis_name="s",
                                  num_cores=NC, num_subcores=NS)
    out_type = [jax.ShapeDtypeStruct((NC, N_PAD, d), jnp.float32)]
    scratch = [
        pltpu.VMEM((CH,), jnp.int32),
        pltpu.VMEM((CH,), jnp.int32),
        pltpu.VMEM((CH, d), jnp.float32),
        pltpu.VMEM_SHARED((N_PAD, d), jnp.float32),
        pltpu.SemaphoreType.DMA,
    ]
    if with_count:
        out_type.append(jax.ShapeDtypeStruct((NC, N_PAD, 16), jnp.float32))
        scratch += [
            pltpu.VMEM((CH, 16), jnp.float32),
            pltpu.VMEM_SHARED((N_PAD, 16), jnp.float32),
        ]

    zrows = jnp.zeros((N_PAD, d), jnp.float32)
    zcnt = jnp.zeros((N_PAD, 16), jnp.float32)
    ones = jnp.ones((CH, 16), jnp.float32)

    def body(x_hbm, src_hbm, dst_hbm, zrows_hbm, zcnt_hbm, ones_hbm,
             out_hbm, cnt_hbm, src_v, dst_v, rows_v, acc_sh, sem,
             ones_v=None, cnt_sh=None):
        c = lax.axis_index("c")
        s = lax.axis_index("s")
        wid = c * NS + s
        r0 = s * RPT
        pltpu.sync_copy(zrows_hbm.at[pl.ds(r0, RPT)], acc_sh.at[pl.ds(r0, RPT)])
        if with_count:
            pltpu.sync_copy(zcnt_hbm.at[pl.ds(r0, RPT)],
                            cnt_sh.at[pl.ds(r0, RPT)])
            pltpu.sync_copy(ones_hbm, ones_v)
        plsc.subcore_barrier()

        base = wid * EPW

        def step(j, carry):
            off = base + j * CH
            pltpu.sync_copy(src_hbm.at[pl.ds(off, CH)], src_v)
            pltpu.sync_copy(dst_hbm.at[pl.ds(off, CH)], dst_v)
            pltpu.async_copy(x_hbm.at[src_v], rows_v, sem).wait()
            pltpu.sync_copy(rows_v, acc_sh.at[dst_v], add=True)
            if with_count:
                pltpu.sync_copy(ones_v, cnt_sh.at[dst_v], add=True)
            return carry

        lax.fori_loop(0, EPW // CH, step, 0)
        plsc.subcore_barrier()
        pltpu.sync_copy(acc_sh.at[pl.ds(r0, RPT)],
                        out_hbm.at[c, pl.ds(r0, RPT)])
        if with_count:
            pltpu.sync_copy(cnt_sh.at[pl.ds(r0, RPT)],
                            cnt_hbm.at[c, pl.ds(r0, RPT)])

    cp = pltpu.CompilerParams(use_tc_tiling_on_sc=False)
    if with_count:
        k = pl.kernel(body, out_type=out_type, mesh=mesh,
                      scratch_types=scratch, compiler_params=cp)
        res = k(x, src, dst, zrows, zcnt, ones)
    else:
        def body_nc(x_hbm, src_hbm, dst_hbm, zrows_hbm, zcnt_hbm, ones_hbm,
                    out_hbm, src_v, dst_v, rows_v, acc_sh, sem):
            body(x_hbm, src_hbm, dst_hbm, zrows_hbm, zcnt_hbm, ones_hbm,
                 out_hbm, None, src_v, dst_v, rows_v, acc_sh, sem)
        k = pl.kernel(body_nc, out_type=out_type, mesh=mesh,
                      scratch_types=scratch, compiler_params=cp)
        res = k(x, src, dst, zrows, zcnt, ones)
    if not isinstance(res, (list, tuple)):
        res = (res,)
    return tuple(res)


def _edge_gather(h, src, dst):
    mesh = plsc.VectorSubcoreMesh(core_axis_name="c", subcore_axis_name="s",
                                  num_cores=NC, num_subcores=NS)
    out_type = [jax.ShapeDtypeStruct((E, H), jnp.float32),
                jax.ShapeDtypeStruct((E, H), jnp.float32)]
    scratch = [
        pltpu.VMEM((CH,), jnp.int32),
        pltpu.VMEM((CH,), jnp.int32),
        pltpu.VMEM((CH, H), jnp.float32),
        pltpu.VMEM((CH, H), jnp.float32),
        pltpu.SemaphoreType.DMA,
        pltpu.SemaphoreType.DMA,
    ]

    def body(h_hbm, src_hbm, dst_hbm, outs_hbm, outd_hbm,
             src_v, dst_v, rs_v, rd_v, sem_a, sem_b):
        c = lax.axis_index("c")
        s = lax.axis_index("s")
        wid = c * NS + s
        base = wid * EPW

        def step(j, carry):
            off = base + j * CH
            pltpu.sync_copy(src_hbm.at[pl.ds(off, CH)], src_v)
            pltpu.sync_copy(dst_hbm.at[pl.ds(off, CH)], dst_v)
            ga = pltpu.async_copy(h_hbm.at[src_v], rs_v, sem_a)
            gb = pltpu.async_copy(h_hbm.at[dst_v], rd_v, sem_b)
            ga.wait()
            pltpu.sync_copy(rs_v, outs_hbm.at[pl.ds(off, CH)])
            gb.wait()
            pltpu.sync_copy(rd_v, outd_hbm.at[pl.ds(off, CH)])
            return carry

        lax.fori_loop(0, EPW // CH, step, 0)

    k = pl.kernel(body, out_type=out_type, mesh=mesh, scratch_types=scratch,
                  compiler_params=pltpu.CompilerParams(use_tc_tiling_on_sc=False))
    return k(h, src, dst)


def _combine_kernel(sums_ref, cnt_ref, x_ref, wl_ref, wr_ref, b_ref, o_ref):
    s = sums_ref[0] + sums_ref[1]
    deg = cnt_ref[0, :, 0:1] + cnt_ref[1, :, 0:1]
    agg = s * (1.0 / jnp.maximum(deg, 1.0))
    o = (jnp.dot(agg, wl_ref[...], preferred_element_type=jnp.float32)
         + jnp.dot(x_ref[...], wr_ref[...], preferred_element_type=jnp.float32)
         + b_ref[...])
    o_ref[...] = jnp.maximum(o, 0.0)


def _sage_combine(sums, cnt, x, wlT, wrT, b):
    d = x.shape[1]
    R = 1000
    grid = N // R
    return pl.pallas_call(
        _combine_kernel,
        grid=(grid,),
        in_specs=[
            pl.BlockSpec((NC, R, d), lambda i: (0, i, 0)),
            pl.BlockSpec((NC, R, 16), lambda i: (0, i, 0)),
            pl.BlockSpec((R, d), lambda i: (i, 0)),
            pl.BlockSpec((d, H), lambda i: (0, 0)),
            pl.BlockSpec((d, H), lambda i: (0, 0)),
            pl.BlockSpec((1, H), lambda i: (0, 0)),
        ],
        out_specs=pl.BlockSpec((R, H), lambda i: (i, 0)),
        out_shape=jax.ShapeDtypeStruct((N, H), jnp.float32),
    )(sums, cnt, x, wlT, wrT, b)


def _combine_bn_kernel(sums_ref, cnt_ref, h_ref, wl_ref, wr_ref, b_ref,
                       g_ref, be_ref, o_ref):
    s = sums_ref[0, :N, :] + sums_ref[1, :N, :]
    deg = cnt_ref[0, :N, 0:1] + cnt_ref[1, :N, 0:1]
    agg = s * (1.0 / jnp.maximum(deg, 1.0))
    h2 = (jnp.dot(agg, wl_ref[...], preferred_element_type=jnp.float32)
          + jnp.dot(h_ref[...], wr_ref[...], preferred_element_type=jnp.float32)
          + b_ref[...])
    h2 = jnp.maximum(h2, 0.0)
    mean = jnp.mean(h2, axis=0, keepdims=True)
    cen = h2 - mean
    var = jnp.mean(cen * cen, axis=0, keepdims=True)
    scale = g_ref[...] * lax.rsqrt(var + 1e-5)
    o_ref[...] = cen * scale + be_ref[...]


def _sage_combine_bn(sums, cnt, h1, wlT, wrT, b, g, be):
    args = (sums, cnt, h1, wlT, wrT, b, g, be)
    return pl.pallas_call(
        _combine_bn_kernel,
        grid=(1,),
        in_specs=[pl.BlockSpec(a.shape, lambda i, r=a.ndim: (0,) * r)
                  for a in args],
        out_specs=pl.BlockSpec((N, H), lambda i: (0, 0)),
        out_shape=jax.ShapeDtypeStruct((N, H), jnp.float32),
    )(*args)


def _edge_mlp_kernel(hs_ref, hd_ref, ea_ref, w1s_ref, w1d_ref, w1e_ref,
                     b1_ref, w2_ref, b2_ref, w3_ref, b3_ref, o_ref):
    z1 = (jnp.dot(hs_ref[...], w1s_ref[...], preferred_element_type=jnp.float32)
          + jnp.dot(hd_ref[...], w1d_ref[...], preferred_element_type=jnp.float32)
          + jnp.dot(ea_ref[...], w1e_ref[...], preferred_element_type=jnp.float32)
          + b1_ref[...])
    z1 = jnp.maximum(z1, 0.0)
    z2 = jnp.dot(z1, w2_ref[...], preferred_element_type=jnp.float32) + b2_ref[...]
    z2 = jnp.maximum(z2, 0.0)
    o = jnp.sum(z2 * w3_ref[...], axis=1) + b3_ref[0, 0]
    o_ref[...] = o.reshape(o_ref.shape)


def _edge_mlp(hs, hd, ea, w1sT, w1dT, w1eT, b1, w2T, b2, w3, b3):
    EB = 1280
    grid = E // EB
    F1 = w1sT.shape[1]
    return pl.pallas_call(
        _edge_mlp_kernel,
        grid=(grid,),
        in_specs=[
            pl.BlockSpec((EB, H), lambda i: (i, 0)),
            pl.BlockSpec((EB, H), lambda i: (i, 0)),
            pl.BlockSpec((EB, EA), lambda i: (i, 0)),
            pl.BlockSpec((H, F1), lambda i: (0, 0)),
            pl.BlockSpec((H, F1), lambda i: (0, 0)),
            pl.BlockSpec((EA, F1), lambda i: (0, 0)),
            pl.BlockSpec((1, F1), lambda i: (0, 0)),
            pl.BlockSpec((F1, H), lambda i: (0, 0)),
            pl.BlockSpec((1, H), lambda i: (0, 0)),
            pl.BlockSpec((1, H), lambda i: (0, 0)),
            pl.BlockSpec((1, 1), lambda i: (0, 0)),
        ],
        out_specs=pl.BlockSpec((1, 8, EB // 8), lambda i: (i, 0, 0)),
        out_shape=jax.ShapeDtypeStruct((grid, 8, EB // 8), jnp.float32),
    )(hs, hd, ea, w1sT, w1dT, w1eT, b1, w2T, b2, w3, b3).reshape(E)


def kernel(x, full_ei, edge_index_batch, edge_attr_batch,
           W_l1, b_l1, W_r1, W_l2, b_l2, W_r2, gamma, beta,
           W1, b1, W2, b2, W3, b3):
    src1 = full_ei[0].astype(jnp.int32)
    dst1 = full_ei[1].astype(jnp.int32)
    src2 = edge_index_batch[0].astype(jnp.int32)
    dst2 = edge_index_batch[1].astype(jnp.int32)

    sums1, cnt = _segsum(x, src1, dst1, D, with_count=True)
    h1 = _sage_combine(sums1, cnt, x, W_l1.T, W_r1.T, b_l1[None])

    (sums2,) = _segsum(h1, src1, dst1, H, with_count=False)
    hn = _sage_combine_bn(sums2, cnt, h1, W_l2.T, W_r2.T, b_l2[None],
                          gamma[None], beta[None])

    hs, hd = _edge_gather(hn, src2, dst2)
    out = _edge_mlp(hs, hd, edge_attr_batch,
                    W1[:, :H].T, W1[:, H:2 * H].T, W1[:, 2 * H:].T,
                    b1[None], W2.T, b2[None], W3, b3[None])
    return out


# speedup vs baseline: 4.4301x; 1.3260x over previous
import functools

import jax
import jax.numpy as jnp
from jax import lax
from jax.experimental import pallas as pl
from jax.experimental.pallas import tpu as pltpu
from jax.experimental.pallas import tpu_sc as plsc

N = 10000
E = 320000
D = 128
H = 96
EA = 16

NC = 2
NS = 16
NW = NC * NS
CH = 80
EPW = E // NW
IT = EPW // CH
N_PAD = ((N + 1 + NS * 8 - 1) // (NS * 8)) * (NS * 8)
RPT = N_PAD // NS


def _segsum(x, src, dst, d, with_count):
    mesh = plsc.VectorSubcoreMesh(core_axis_name="c", subcore_axis_name="s",
                                  num_cores=NC, num_subcores=NS)
    out_type = [jax.ShapeDtypeStruct((NC, N_PAD, d), jnp.float32)]
    scratch = [
        pltpu.VMEM((2, CH), jnp.int32),
        pltpu.VMEM((2, CH), jnp.int32),
        pltpu.VMEM((CH, d), jnp.float32),
        pltpu.VMEM((CH, d), jnp.float32),
        pltpu.SemaphoreType.DMA,
        pltpu.SemaphoreType.DMA,
        pltpu.SemaphoreType.DMA,
        pltpu.SemaphoreType.DMA,
        pltpu.VMEM_SHARED((N_PAD, d), jnp.float32),
    ]
    if with_count:
        out_type.append(jax.ShapeDtypeStruct((NC, N_PAD, 16), jnp.float32))
        scratch += [
            pltpu.VMEM((CH, 16), jnp.float32),
            pltpu.VMEM_SHARED((N_PAD, 16), jnp.float32),
        ]

    zrows = jnp.zeros((N_PAD, d), jnp.float32)
    zcnt = jnp.zeros((N_PAD, 16), jnp.float32)
    ones = jnp.ones((CH, 16), jnp.float32)
    ei4 = jnp.pad(
        jnp.stack([src.reshape(NW, IT, CH), dst.reshape(NW, IT, CH)], axis=2),
        ((0, 0), (0, 1), (0, 0), (0, 0)))

    def body(x_hbm, ei_hbm, zrows_hbm, zcnt_hbm, ones_hbm,
             out_hbm, cnt_hbm, idx_a, idx_b, rows_a, rows_b,
             sem_a, sem_b, sem_ia, sem_ib, ones_v=None, cnt_sh=None,
             acc_sh=None):
        c = lax.axis_index("c")
        s = lax.axis_index("s")
        wid = c * NS + s
        r0 = s * RPT
        pltpu.sync_copy(ei_hbm.at[wid, 0], idx_a)
        pltpu.sync_copy(zrows_hbm.at[pl.ds(r0, RPT)], acc_sh.at[pl.ds(r0, RPT)])
        if with_count:
            pltpu.sync_copy(zcnt_hbm.at[pl.ds(r0, RPT)],
                            cnt_sh.at[pl.ds(r0, RPT)])
            pltpu.sync_copy(ones_hbm, ones_v)
        plsc.subcore_barrier()

        def scat(buf, idx_v):
            pltpu.sync_copy(buf, acc_sh.at[idx_v.at[1]], add=True)
            if with_count:
                pltpu.sync_copy(ones_v, cnt_sh.at[idx_v.at[1]], add=True)

        def gat(idx_v, buf, sem):
            pltpu.async_copy(x_hbm.at[idx_v.at[0]], buf, sem)

        def gat_wait(idx_v, buf, sem):
            pltpu.make_async_copy(x_hbm.at[idx_v.at[0]], buf, sem).wait()

        def idx_load(j, idx_v, sem):
            pltpu.async_copy(ei_hbm.at[wid, j], idx_v, sem)

        def idx_wait(j, idx_v, sem):
            pltpu.make_async_copy(ei_hbm.at[wid, j], idx_v, sem).wait()

        gat(idx_a, rows_a, sem_a)
        idx_load(1, idx_b, sem_ib)

        def step(i, carry):
            a = 2 * i
            b = a + 1
            gat_wait(idx_a, rows_a, sem_a)
            idx_wait(b, idx_b, sem_ib)
            gat(idx_b, rows_b, sem_b)
            scat(rows_a, idx_a)
            idx_load(a + 2, idx_a, sem_ia)
            gat_wait(idx_b, rows_b, sem_b)
            idx_wait(a + 2, idx_a, sem_ia)
            gat(idx_a, rows_a, sem_a)
            scat(rows_b, idx_b)
            idx_load(b + 2, idx_b, sem_ib)
            return carry

        lax.fori_loop(0, (IT - 1) // 2, step, 0)
        gat_wait(idx_a, rows_a, sem_a)
        scat(rows_a, idx_a)
        idx_wait(IT, idx_b, sem_ib)

        plsc.subcore_barrier()
        pltpu.sync_copy(acc_sh.at[pl.ds(r0, RPT)],
                        out_hbm.at[c, pl.ds(r0, RPT)])
        if with_count:
            pltpu.sync_copy(cnt_sh.at[pl.ds(r0, RPT)],
                            cnt_hbm.at[c, pl.ds(r0, RPT)])

    cp = pltpu.CompilerParams(use_tc_tiling_on_sc=False)
    if with_count:
        def body_wc(x_hbm, ei_hbm, zrows_hbm, zcnt_hbm, ones_hbm,
                    out_hbm, cnt_hbm, idx_a, idx_b, rows_a, rows_b,
                    sem_a, sem_b, sem_ia, sem_ib, ones_v, cnt_sh, acc_sh):
            body(x_hbm, ei_hbm, zrows_hbm, zcnt_hbm, ones_hbm,
                 out_hbm, cnt_hbm, idx_a, idx_b, rows_a, rows_b,
                 sem_a, sem_b, sem_ia, sem_ib, ones_v, cnt_sh, acc_sh)
        scratch_wc = scratch[:8] + [scratch[9], scratch[10], scratch[8]]
        k = pl.kernel(body_wc, out_type=out_type, mesh=mesh,
                      scratch_types=scratch_wc, compiler_params=cp)
        res = k(x, ei4, zrows, zcnt, ones)
    else:
        def body_nc(x_hbm, ei_hbm, zrows_hbm, zcnt_hbm, ones_hbm,
                    out_hbm, idx_a, idx_b, rows_a, rows_b,
                    sem_a, sem_b, sem_ia, sem_ib, acc_sh):
            body(x_hbm, ei_hbm, zrows_hbm, zcnt_hbm, ones_hbm,
                 out_hbm, None, idx_a, idx_b, rows_a, rows_b,
                 sem_a, sem_b, sem_ia, sem_ib, None, None, acc_sh)
        k = pl.kernel(body_nc, out_type=out_type, mesh=mesh,
                      scratch_types=scratch, compiler_params=cp)
        res = k(x, ei4, zrows, zcnt, ones)
    if not isinstance(res, (list, tuple)):
        res = (res,)
    return tuple(res)


def _edge_gather(h, src, dst):
    mesh = plsc.VectorSubcoreMesh(core_axis_name="c", subcore_axis_name="s",
                                  num_cores=NC, num_subcores=NS)
    out_type = [jax.ShapeDtypeStruct((E, H), jnp.float32),
                jax.ShapeDtypeStruct((E, H), jnp.float32)]
    scratch = [
        pltpu.VMEM((IT, CH), jnp.int32),
        pltpu.VMEM((IT, CH), jnp.int32),
        pltpu.VMEM((CH, H), jnp.float32),
        pltpu.VMEM((CH, H), jnp.float32),
        pltpu.VMEM((CH, H), jnp.float32),
        pltpu.VMEM((CH, H), jnp.float32),
        pltpu.SemaphoreType.DMA,
        pltpu.SemaphoreType.DMA,
        pltpu.SemaphoreType.DMA,
        pltpu.SemaphoreType.DMA,
    ]
    src3 = src.reshape(NW, IT, CH)
    dst3 = dst.reshape(NW, IT, CH)

    def body(h_hbm, src_hbm, dst_hbm, outs_hbm, outd_hbm,
             srcs_v, dsts_v, sa, sb, da, db, sem_sa, sem_sb, sem_da, sem_db):
        c = lax.axis_index("c")
        s = lax.axis_index("s")
        wid = c * NS + s
        base = wid * EPW
        pltpu.sync_copy(src_hbm.at[wid], srcs_v)
        pltpu.sync_copy(dst_hbm.at[wid], dsts_v)

        def gat(idx_v, j, buf, sem):
            return pltpu.async_copy(h_hbm.at[idx_v.at[j]], buf, sem)

        def gat_wait(idx_v, j, buf, sem):
            pltpu.make_async_copy(h_hbm.at[idx_v.at[j]], buf, sem).wait()

        def wr(bufs, bufd, j):
            off = base + j * CH
            pltpu.sync_copy(bufs, outs_hbm.at[pl.ds(off, CH)])
            pltpu.sync_copy(bufd, outd_hbm.at[pl.ds(off, CH)])

        gat(srcs_v, 0, sa, sem_sa)
        gat(dsts_v, 0, da, sem_da)

        def step(i, carry):
            a = 2 * i
            b = a + 1
            gat_wait(srcs_v, a, sa, sem_sa)
            gat_wait(dsts_v, a, da, sem_da)
            gat(srcs_v, b, sb, sem_sb)
            gat(dsts_v, b, db, sem_db)
            wr(sa, da, a)
            gat_wait(srcs_v, b, sb, sem_sb)
            gat_wait(dsts_v, b, db, sem_db)
            gat(srcs_v, a + 2, sa, sem_sa)
            gat(dsts_v, a + 2, da, sem_da)
            wr(sb, db, b)
            return carry

        lax.fori_loop(0, (IT - 1) // 2, step, 0)
        gat_wait(srcs_v, IT - 1, sa, sem_sa)
        gat_wait(dsts_v, IT - 1, da, sem_da)
        wr(sa, da, IT - 1)

    k = pl.kernel(body, out_type=out_type, mesh=mesh, scratch_types=scratch,
                  compiler_params=pltpu.CompilerParams(use_tc_tiling_on_sc=False))
    return k(h, src3, dst3)


def _combine_kernel(sums_ref, cnt_ref, x_ref, wl_ref, wr_ref, b_ref, o_ref):
    s = sums_ref[0] + sums_ref[1]
    deg = cnt_ref[0, :, 0:1] + cnt_ref[1, :, 0:1]
    agg = s * (1.0 / jnp.maximum(deg, 1.0))
    o = (jnp.dot(agg, wl_ref[...], preferred_element_type=jnp.float32)
         + jnp.dot(x_ref[...], wr_ref[...], preferred_element_type=jnp.float32)
         + b_ref[...])
    o_ref[...] = jnp.maximum(o, 0.0)


def _sage_combine(sums, cnt, x, wlT, wrT, b):
    d = x.shape[1]
    R = 1000
    grid = N // R
    return pl.pallas_call(
        _combine_kernel,
        grid=(grid,),
        in_specs=[
            pl.BlockSpec((NC, R, d), lambda i: (0, i, 0)),
            pl.BlockSpec((NC, R, 16), lambda i: (0, i, 0)),
            pl.BlockSpec((R, d), lambda i: (i, 0)),
            pl.BlockSpec((d, H), lambda i: (0, 0)),
            pl.BlockSpec((d, H), lambda i: (0, 0)),
            pl.BlockSpec((1, H), lambda i: (0, 0)),
        ],
        out_specs=pl.BlockSpec((R, H), lambda i: (i, 0)),
        out_shape=jax.ShapeDtypeStruct((N, H), jnp.float32),
    )(sums, cnt, x, wlT, wrT, b)


def _combine_bn_kernel(sums_ref, cnt_ref, h_ref, wl_ref, wr_ref, b_ref,
                       g_ref, be_ref, o_ref):
    s = sums_ref[0, :N, :] + sums_ref[1, :N, :]
    deg = cnt_ref[0, :N, 0:1] + cnt_ref[1, :N, 0:1]
    agg = s * (1.0 / jnp.maximum(deg, 1.0))
    h2 = (jnp.dot(agg, wl_ref[...], preferred_element_type=jnp.float32)
          + jnp.dot(h_ref[...], wr_ref[...], preferred_element_type=jnp.float32)
          + b_ref[...])
    h2 = jnp.maximum(h2, 0.0)
    mean = jnp.mean(h2, axis=0, keepdims=True)
    cen = h2 - mean
    var = jnp.mean(cen * cen, axis=0, keepdims=True)
    scale = g_ref[...] * lax.rsqrt(var + 1e-5)
    o_ref[...] = cen * scale + be_ref[...]


def _sage_combine_bn(sums, cnt, h1, wlT, wrT, b, g, be):
    args = (sums, cnt, h1, wlT, wrT, b, g, be)
    return pl.pallas_call(
        _combine_bn_kernel,
        grid=(1,),
        in_specs=[pl.BlockSpec(a.shape, lambda i, r=a.ndim: (0,) * r)
                  for a in args],
        out_specs=pl.BlockSpec((N, H), lambda i: (0, 0)),
        out_shape=jax.ShapeDtypeStruct((N, H), jnp.float32),
    )(*args)


def _edge_mlp_kernel(hs_ref, hd_ref, ea_ref, w1s_ref, w1d_ref, w1e_ref,
                     b1_ref, w2_ref, b2_ref, w3_ref, b3_ref, o_ref):
    z1 = (jnp.dot(hs_ref[...], w1s_ref[...], preferred_element_type=jnp.float32)
          + jnp.dot(hd_ref[...], w1d_ref[...], preferred_element_type=jnp.float32)
          + jnp.dot(ea_ref[...], w1e_ref[...], preferred_element_type=jnp.float32)
          + b1_ref[...])
    z1 = jnp.maximum(z1, 0.0)
    z2 = jnp.dot(z1, w2_ref[...], preferred_element_type=jnp.float32) + b2_ref[...]
    z2 = jnp.maximum(z2, 0.0)
    o = jnp.sum(z2 * w3_ref[...], axis=1) + b3_ref[0, 0]
    o_ref[...] = o.reshape(o_ref.shape)


def _edge_mlp(hs, hd, ea, w1sT, w1dT, w1eT, b1, w2T, b2, w3, b3):
    EB = 1280
    grid = E // EB
    F1 = w1sT.shape[1]
    return pl.pallas_call(
        _edge_mlp_kernel,
        grid=(grid,),
        in_specs=[
            pl.BlockSpec((EB, H), lambda i: (i, 0)),
            pl.BlockSpec((EB, H), lambda i: (i, 0)),
            pl.BlockSpec((EB, EA), lambda i: (i, 0)),
            pl.BlockSpec((H, F1), lambda i: (0, 0)),
            pl.BlockSpec((H, F1), lambda i: (0, 0)),
            pl.BlockSpec((EA, F1), lambda i: (0, 0)),
            pl.BlockSpec((1, F1), lambda i: (0, 0)),
            pl.BlockSpec((F1, H), lambda i: (0, 0)),
            pl.BlockSpec((1, H), lambda i: (0, 0)),
            pl.BlockSpec((1, H), lambda i: (0, 0)),
            pl.BlockSpec((1, 1), lambda i: (0, 0)),
        ],
        out_specs=pl.BlockSpec((1, 8, EB // 8), lambda i: (i, 0, 0)),
        out_shape=jax.ShapeDtypeStruct((grid, 8, EB // 8), jnp.float32),
    )(hs, hd, ea, w1sT, w1dT, w1eT, b1, w2T, b2, w3, b3).reshape(E)


def kernel(x, full_ei, edge_index_batch, edge_attr_batch,
           W_l1, b_l1, W_r1, W_l2, b_l2, W_r2, gamma, beta,
           W1, b1, W2, b2, W3, b3):
    src1 = full_ei[0].astype(jnp.int32)
    dst1 = full_ei[1].astype(jnp.int32)
    src2 = edge_index_batch[0].astype(jnp.int32)
    dst2 = edge_index_batch[1].astype(jnp.int32)

    sums1, cnt = _segsum(x, src1, dst1, D, with_count=True)
    h1 = _sage_combine(sums1, cnt, x, W_l1.T, W_r1.T, b_l1[None])

    (sums2,) = _segsum(h1, src1, dst1, H, with_count=False)
    hn = _sage_combine_bn(sums2, cnt, h1, W_l2.T, W_r2.T, b_l2[None],
                          gamma[None], beta[None])

    hs, hd = _edge_gather(hn, src2, dst2)
    out = _edge_mlp(hs, hd, edge_attr_batch,
                    W1[:, :H].T, W1[:, H:2 * H].T, W1[:, 2 * H:].T,
                    b1[None], W2.T, b2[None], W3, b3[None])
    return out
